# Initial kernel scaffold; baseline (speedup 1.0000x reference)
import jax
import jax.numpy as jnp
from jax.experimental import pallas as pl


def kernel(inputs, token_table, pos_table):
    raise NotImplementedError("write your pallas kernel here")


# speedup vs baseline: 3.3300x; 3.3300x over previous
import jax
import jax.numpy as jnp
from jax import lax
from jax.experimental import pallas as pl
from jax.experimental.pallas import tpu as pltpu
from jax.experimental.pallas import tpu_sc as plsc

NC = 2
NS = 16
NW = NC * NS

G = 100
GPC = 4
CH = G * GPC


def _body(seq, emb, nchunk, idx_hbm, tok_hbm, pos_hbm, out_hbm,
          idx_v, tmpl_v, buf_v, sem):
    wid = lax.axis_index("s") * NC + lax.axis_index("c")
    cbase = wid * nchunk

    for r in range(CH // seq):
        pltpu.sync_copy(pos_hbm, tmpl_v.at[pl.ds(r * seq, seq)])

    def chunk_body(c, carry):
        cid = cbase + c
        pltpu.sync_copy(idx_hbm.at[cid], idx_v)
        cps = [
            pltpu.async_copy(tok_hbm.at[idx_v.at[j]],
                             buf_v.at[pl.ds(j * G, G)], sem)
            for j in range(GPC)
        ]
        for cp in cps:
            cp.wait()

        def row_body(r, rcarry):
            for j in range(emb // 16):
                sl = pl.ds(j * 16, 16)
                plsc.addupdate(buf_v.at[r, sl], tmpl_v[r, sl])
            return rcarry
        lax.fori_loop(0, CH, row_body, 0, unroll=2)

        off = pl.multiple_of(cid * CH, CH)
        pltpu.sync_copy(buf_v, out_hbm.at[pl.ds(off, CH)])
        return carry

    lax.fori_loop(0, nchunk, chunk_body, 0)


@jax.jit
def kernel(inputs, token_table, pos_table):
    batch, seq = inputs.shape
    emb = token_table.shape[1]
    rows = batch * seq
    nchunk = rows // NW // CH

    idx3 = inputs.astype(jnp.int32).reshape(rows // CH, GPC, G)
    mesh = plsc.VectorSubcoreMesh(core_axis_name="c", subcore_axis_name="s")
    body = lambda *refs: _body(seq, emb, nchunk, *refs)
    out = pl.kernel(
        body,
        out_type=jax.ShapeDtypeStruct((rows, emb), jnp.float32),
        mesh=mesh,
        compiler_params=pltpu.CompilerParams(use_tc_tiling_on_sc=False),
        scratch_types=[
            pltpu.VMEM((GPC, G), jnp.int32),
            pltpu.VMEM((CH, emb), jnp.float32),
            pltpu.VMEM((CH, emb), jnp.float32),
            pltpu.SemaphoreType.DMA,
        ],
    )(idx3, token_table, pos_table)
    return out.reshape(batch, seq, emb)


# speedup vs baseline: 4.2151x; 1.2658x over previous
import jax
import jax.numpy as jnp
from jax import lax
from jax.experimental import pallas as pl
from jax.experimental.pallas import tpu as pltpu
from jax.experimental.pallas import tpu_sc as plsc

NC = 2
NS = 16
NW = NC * NS

G = 100
GPC = 2
CH = G * GPC
NBUF = 4


def _body(seq, emb, nchunk, idx_hbm, tok_hbm, pos_hbm, out_hbm,
          idx_all, tmpl_v, bufs, gsems, wsems):
    wid = lax.axis_index("s") * NC + lax.axis_index("c")
    rbase = wid * (nchunk * GPC)
    obase = wid * (nchunk * CH)

    pltpu.sync_copy(idx_hbm.at[pl.ds(rbase, nchunk * GPC)], idx_all)
    pltpu.sync_copy(pos_hbm, tmpl_v)

    def fire_gathers(u, c):
        for j in range(GPC):
            pltpu.async_copy(tok_hbm.at[idx_all.at[c * GPC + j]],
                             bufs[u].at[pl.ds(j * G, G)], gsems[u])

    def wait_gathers(u):
        for j in range(GPC):
            pltpu.make_async_copy(tok_hbm.at[idx_all.at[0]],
                                  bufs[u].at[pl.ds(j * G, G)],
                                  gsems[u]).wait()

    def wait_writeback(u):
        pltpu.make_async_copy(bufs[u], out_hbm.at[pl.ds(0, CH)],
                              wsems[u]).wait()

    fire_gathers(0, 0)
    fire_gathers(1, 1)

    def iter_body(k, carry):
        for u in range(NBUF):
            c = k * NBUF + u
            w = (u + 2) % NBUF
            wait_gathers(u)
            if u < 2:
                @pl.when(k > 0)
                def _():
                    wait_writeback(w)
                fire_gathers(w, c + 2)
            else:
                wait_writeback(w)
                @pl.when(c + 2 < nchunk)
                def _():
                    fire_gathers(w, c + 2)

            def row_body(r, rcarry):
                for j in range(emb // 16):
                    sl = pl.ds(j * 16, 16)
                    plsc.addupdate(bufs[u].at[r, sl], tmpl_v[r, sl])
                return rcarry
            lax.fori_loop(0, CH, row_body, 0, unroll=4)

            off = pl.multiple_of(obase + c * CH, 8)
            pltpu.async_copy(bufs[u], out_hbm.at[pl.ds(off, CH)], wsems[u])
        return carry

    lax.fori_loop(0, nchunk // NBUF, iter_body, 0)
    for u in (2, 3):
        wait_writeback(u)


@jax.jit
def kernel(inputs, token_table, pos_table):
    batch, seq = inputs.shape
    emb = token_table.shape[1]
    rows = batch * seq
    nchunk = rows // NW // CH

    idx2 = inputs.astype(jnp.int32).reshape(rows // G, G)
    mesh = plsc.VectorSubcoreMesh(core_axis_name="c", subcore_axis_name="s")
    body = lambda *refs: _body(seq, emb, nchunk, *refs)
    out = pl.kernel(
        body,
        out_type=jax.ShapeDtypeStruct((rows, emb), jnp.float32),
        mesh=mesh,
        compiler_params=pltpu.CompilerParams(use_tc_tiling_on_sc=False),
        scratch_types=[
            pltpu.VMEM((rows // NW // G, G), jnp.int32),
            pltpu.VMEM((seq, emb), jnp.float32),
            [pltpu.VMEM((CH, emb), jnp.float32) for _ in range(NBUF)],
            [pltpu.SemaphoreType.DMA for _ in range(NBUF)],
            [pltpu.SemaphoreType.DMA for _ in range(NBUF)],
        ],
    )(idx2, token_table, pos_table)
    return out.reshape(batch, seq, emb)
